# baseline (device time: 192736 ns/iter reference)
import jax
import jax.numpy as jnp
from jax import lax
from jax.experimental import pallas as pl
from jax.experimental.pallas import tpu as pltpu

_DIAG_NO_DOT = False

N_DEV = 4
M_PER = 1024
K = 4096
N_PER = 2048
HALF = 1024
N_STEPS = 8


def kernel(x, w_mat):
    def body(x_ref, w_hbm, out_hbm, stage_hbm, w_bufs, comm_bufs,
             amax_buf, w_sems, send_sems, recv_sems, ax_send_sems,
             ax_recv_sems, own_sems, epi_sems, epo_sems):
        my = lax.axis_index("i")

        barrier = pltpu.get_barrier_semaphore()
        for d in range(1, N_DEV):
            pl.semaphore_signal(
                barrier, inc=1,
                device_id=((my + d) % N_DEV,),
                device_id_type=pl.DeviceIdType.MESH,
            )
        pl.semaphore_wait(barrier, N_DEV - 1)

        def step_cols(t):
            dst = (my + 1 + t // 2) % N_DEV
            h = t % 2
            return dst, h

        def w_copy(t):
            dst, h = step_cols(t)
            col = dst * N_PER + h * HALF
            return [
                pltpu.make_async_copy(
                    w_hbm.at[pl.ds(j * (K // 4), K // 4), pl.ds(col, HALF)],
                    w_bufs.at[t % 2, pl.ds(j * (K // 4), K // 4)],
                    w_sems.at[t % 2, j],
                )
                for j in range(4)
            ]

        for c in w_copy(0):
            c.start()

        sends = {}
        own = []
        amax = jnp.float32(0.0)
        for t in range(N_STEPS):
            if t + 1 < N_STEPS:
                for c in w_copy(t + 1):
                    c.start()
            for c in w_copy(t):
                c.wait()
            if _DIAG_NO_DOT:
                blk = w_bufs[t % 2, pl.ds(0, M_PER)] * 2.0
            else:
                blk = jnp.dot(x_ref[...], w_bufs[t % 2],
                              preferred_element_type=jnp.float32)
            amax = jnp.maximum(amax, jnp.max(jnp.abs(blk)))
            dst, h = step_cols(t)
            if t - 2 in sends:
                sends[t - 2].wait_send()
            comm_bufs[t % 2] = blk.astype(jnp.bfloat16)
            if t < 6:
                rdma = pltpu.make_async_remote_copy(
                    src_ref=comm_bufs.at[t % 2],
                    dst_ref=stage_hbm.at[pl.ds(my * M_PER, M_PER),
                                         pl.ds(h * HALF, HALF)],
                    send_sem=send_sems.at[t],
                    recv_sem=recv_sems.at[t // 2, h],
                    device_id=(dst,),
                    device_id_type=pl.DeviceIdType.MESH,
                )
                rdma.start()
                sends[t] = rdma
            else:
                cp = pltpu.make_async_copy(
                    comm_bufs.at[t % 2],
                    stage_hbm.at[pl.ds(my * M_PER, M_PER),
                                 pl.ds(h * HALF, HALF)],
                    own_sems.at[h],
                )
                cp.start()
                own.append(cp)

        amax_buf[pl.ds(my, 1)] = jnp.full((1, 8, 128), amax, jnp.float32)
        ax_sends = []
        for d in range(1, N_DEV):
            r = pltpu.make_async_remote_copy(
                src_ref=amax_buf.at[pl.ds(my, 1)],
                dst_ref=amax_buf.at[pl.ds(my, 1)],
                send_sem=ax_send_sems.at[d - 1],
                recv_sem=ax_recv_sems.at[d - 1],
                device_id=((my + d) % N_DEV,),
                device_id_type=pl.DeviceIdType.MESH,
            )
            r.start()
            ax_sends.append(r)

        for cp in own:
            cp.wait()

        for d in range(1, N_DEV):
            src = (my - d) % N_DEV
            pltpu.make_async_remote_copy(
                src_ref=amax_buf.at[pl.ds(0, 1)],
                dst_ref=amax_buf.at[pl.ds(src, 1)],
                send_sem=ax_send_sems.at[d - 1],
                recv_sem=ax_recv_sems.at[d - 1],
                device_id=(0,),
                device_id_type=pl.DeviceIdType.MESH,
            ).wait_recv()
        for r in ax_sends:
            r.wait_send()

        for d in range(1, N_DEV):
            src = (my - d) % N_DEV
            for h in range(2):
                pltpu.make_async_remote_copy(
                    src_ref=comm_bufs.at[0],
                    dst_ref=stage_hbm.at[pl.ds(src * M_PER, M_PER),
                                         pl.ds(h * HALF, HALF)],
                    send_sem=send_sems.at[0],
                    recv_sem=recv_sems.at[d - 1, h],
                    device_id=(0,),
                    device_id_type=pl.DeviceIdType.MESH,
                ).wait_recv()

        g = jnp.max(amax_buf[...])
        scale = g / 448.0
        inv = 448.0 / g

        def epi_in(k):
            r, c = k // 2, k % 2
            return [
                pltpu.make_async_copy(
                    stage_hbm.at[pl.ds(r * M_PER + j * 256, 256),
                                 pl.ds(c * HALF, HALF)],
                    comm_bufs.at[k % 2, pl.ds(j * 256, 256)],
                    epi_sems.at[k % 2, j],
                )
                for j in range(4)
            ]

        def epi_out(k):
            return [
                pltpu.make_async_copy(
                    w_bufs.at[k % 2, pl.ds(j * 256, 256)],
                    out_hbm.at[pl.ds((k // 2) * M_PER + j * 256, 256),
                               pl.ds((k % 2) * HALF, HALF)],
                    epo_sems.at[k % 2, j],
                )
                for j in range(4)
            ]

        for c in epi_in(0):
            c.start()
        outs = {}
        for k in range(8):
            for c in epi_in(k):
                c.wait()
            if k - 2 in outs:
                for c in outs[k - 2]:
                    c.wait()
            y = comm_bufs[k % 2].astype(jnp.float32)
            q = jnp.clip(y * inv, -448.0, 448.0)
            q = q.astype(jnp.float8_e4m3fn).astype(jnp.float32)
            w_bufs[k % 2, pl.ds(0, M_PER)] = q * scale
            cps = epi_out(k)
            for c in cps:
                c.start()
            outs[k] = cps
            if k + 1 < 8:
                for c in epi_in(k + 1):
                    c.start()
        for k in (6, 7):
            for c in outs[k]:
                c.wait()

    out, _ = pl.pallas_call(
        body,
        out_shape=(
            jax.ShapeDtypeStruct((N_DEV * M_PER, N_PER), jnp.float32),
            jax.ShapeDtypeStruct((N_DEV * M_PER, N_PER), jnp.bfloat16),
        ),
        in_specs=[
            pl.BlockSpec(memory_space=pltpu.VMEM),
            pl.BlockSpec(memory_space=pl.ANY),
        ],
        out_specs=(
            pl.BlockSpec(memory_space=pl.ANY),
            pl.BlockSpec(memory_space=pl.ANY),
        ),
        scratch_shapes=[
            pltpu.VMEM((2, K, HALF), jnp.float32),
            pltpu.VMEM((2, M_PER, HALF), jnp.bfloat16),
            pltpu.VMEM((N_DEV, 8, 128), jnp.float32),
            pltpu.SemaphoreType.DMA((2, 4)),
            pltpu.SemaphoreType.DMA((6,)),
            pltpu.SemaphoreType.DMA((3, 2)),
            pltpu.SemaphoreType.DMA((3,)),
            pltpu.SemaphoreType.DMA((3,)),
            pltpu.SemaphoreType.DMA((2,)),
            pltpu.SemaphoreType.DMA((2, 4)),
            pltpu.SemaphoreType.DMA((2, 4)),
        ],
        compiler_params=pltpu.CompilerParams(
            collective_id=0,
            vmem_limit_bytes=63 * 1024 * 1024,
        ),
    )(x, w_mat)
    return out


# device time: 182510 ns/iter; 1.0560x vs baseline; 1.0560x over previous
import jax
import jax.numpy as jnp
from jax import lax
from jax.experimental import pallas as pl
from jax.experimental.pallas import tpu as pltpu

_DIAG_NO_DOT = False

N_DEV = 4
M_PER = 1024
K = 4096
N_PER = 2048
HALF = 1024
N_STEPS = 8


def kernel(x, w_mat):
    def body(x_ref, w_hbm, out_hbm, stage_hbm, w_bufs, comm_bufs,
             amax_buf, w_sems, send_sems, recv_sems, ax_send_sems,
             ax_recv_sems, own_sems, epi_sems, epo_sems):
        my = lax.axis_index("i")

        barrier = pltpu.get_barrier_semaphore()
        for d in range(1, N_DEV):
            pl.semaphore_signal(
                barrier, inc=1,
                device_id=((my + d) % N_DEV,),
                device_id_type=pl.DeviceIdType.MESH,
            )
        pl.semaphore_wait(barrier, N_DEV - 1)

        def step_cols(t):
            dst = (my + 1 + t // 2) % N_DEV
            h = t % 2
            return dst, h

        def w_copy(t):
            dst, h = step_cols(t)
            col = dst * N_PER + h * HALF
            return [
                pltpu.make_async_copy(
                    w_hbm.at[pl.ds(j * (K // 4), K // 4), pl.ds(col, HALF)],
                    w_bufs.at[t % 2, pl.ds(j * (K // 4), K // 4)],
                    w_sems.at[t % 2, j],
                )
                for j in range(4)
            ]

        for c in w_copy(0):
            c.start()

        sends = {}
        own = []
        amax = jnp.float32(0.0)
        for t in range(N_STEPS):
            if t + 1 < N_STEPS:
                for c in w_copy(t + 1):
                    c.start()
            for c in w_copy(t):
                c.wait()
            if _DIAG_NO_DOT:
                blk = w_bufs[t % 2, pl.ds(0, M_PER)] * 2.0
            else:
                blk = jnp.dot(x_ref[...], w_bufs[t % 2],
                              preferred_element_type=jnp.float32)
            amax = jnp.maximum(amax, jnp.max(jnp.abs(blk)))
            dst, h = step_cols(t)
            if t - 2 in sends:
                sends[t - 2].wait_send()
            comm_bufs[t % 2] = blk.astype(jnp.bfloat16)
            if t < 6:
                rdma = pltpu.make_async_remote_copy(
                    src_ref=comm_bufs.at[t % 2],
                    dst_ref=stage_hbm.at[pl.ds(my * M_PER, M_PER),
                                         pl.ds(h * HALF, HALF)],
                    send_sem=send_sems.at[t],
                    recv_sem=recv_sems.at[t // 2, h],
                    device_id=(dst,),
                    device_id_type=pl.DeviceIdType.MESH,
                )
                rdma.start()
                sends[t] = rdma
            else:
                cp = pltpu.make_async_copy(
                    comm_bufs.at[t % 2],
                    stage_hbm.at[pl.ds(my * M_PER, M_PER),
                                 pl.ds(h * HALF, HALF)],
                    own_sems.at[h],
                )
                cp.start()
                own.append(cp)

        amax_buf[pl.ds(my, 1)] = jnp.full((1, 8, 128), amax, jnp.float32)
        ax_sends = []
        for d in range(1, N_DEV):
            r = pltpu.make_async_remote_copy(
                src_ref=amax_buf.at[pl.ds(my, 1)],
                dst_ref=amax_buf.at[pl.ds(my, 1)],
                send_sem=ax_send_sems.at[d - 1],
                recv_sem=ax_recv_sems.at[d - 1],
                device_id=((my + d) % N_DEV,),
                device_id_type=pl.DeviceIdType.MESH,
            )
            r.start()
            ax_sends.append(r)

        for cp in own:
            cp.wait()

        for d in range(1, N_DEV):
            src = (my - d) % N_DEV
            pltpu.make_async_remote_copy(
                src_ref=amax_buf.at[pl.ds(0, 1)],
                dst_ref=amax_buf.at[pl.ds(src, 1)],
                send_sem=ax_send_sems.at[d - 1],
                recv_sem=ax_recv_sems.at[d - 1],
                device_id=(0,),
                device_id_type=pl.DeviceIdType.MESH,
            ).wait_recv()
        for r in ax_sends:
            r.wait_send()

        g = jnp.max(amax_buf[...])
        scale = g / 448.0
        inv = 448.0 / g

        def chunk_rows(k):
            return ((my - k // 2) % N_DEV) * M_PER

        def recv_wait(k):
            b, h = k // 2, k % 2
            if b == 0:
                return
            pltpu.make_async_remote_copy(
                src_ref=comm_bufs.at[0],
                dst_ref=stage_hbm.at[pl.ds(chunk_rows(k), M_PER),
                                     pl.ds(h * HALF, HALF)],
                send_sem=send_sems.at[0],
                recv_sem=recv_sems.at[b - 1, h],
                device_id=(0,),
                device_id_type=pl.DeviceIdType.MESH,
            ).wait_recv()

        def epi_in(k):
            return [
                pltpu.make_async_copy(
                    stage_hbm.at[pl.ds(chunk_rows(k) + j * 256, 256),
                                 pl.ds((k % 2) * HALF, HALF)],
                    comm_bufs.at[k % 2, pl.ds(j * 256, 256)],
                    epi_sems.at[k % 2, j],
                )
                for j in range(4)
            ]

        def epi_out(k):
            return [
                pltpu.make_async_copy(
                    w_bufs.at[k % 2, pl.ds(j * 256, 256)],
                    out_hbm.at[pl.ds(chunk_rows(k) + j * 256, 256),
                               pl.ds((k % 2) * HALF, HALF)],
                    epo_sems.at[k % 2, j],
                )
                for j in range(4)
            ]

        recv_wait(0)
        for c in epi_in(0):
            c.start()
        outs = {}
        for k in range(8):
            if k + 1 < 8:
                recv_wait(k + 1)
                for c in epi_in(k + 1):
                    c.start()
            for c in epi_in(k):
                c.wait()
            if k - 2 in outs:
                for c in outs[k - 2]:
                    c.wait()
            y = comm_bufs[k % 2].astype(jnp.float32)
            q = jnp.clip(y * inv, -448.0, 448.0)
            q = q.astype(jnp.float8_e4m3fn).astype(jnp.float32)
            w_bufs[k % 2, pl.ds(0, M_PER)] = q * scale
            cps = epi_out(k)
            for c in cps:
                c.start()
            outs[k] = cps
        for k in (6, 7):
            for c in outs[k]:
                c.wait()

    out, _ = pl.pallas_call(
        body,
        out_shape=(
            jax.ShapeDtypeStruct((N_DEV * M_PER, N_PER), jnp.float32),
            jax.ShapeDtypeStruct((N_DEV * M_PER, N_PER), jnp.bfloat16),
        ),
        in_specs=[
            pl.BlockSpec(memory_space=pltpu.VMEM),
            pl.BlockSpec(memory_space=pl.ANY),
        ],
        out_specs=(
            pl.BlockSpec(memory_space=pl.ANY),
            pl.BlockSpec(memory_space=pl.ANY),
        ),
        scratch_shapes=[
            pltpu.VMEM((2, K, HALF), jnp.float32),
            pltpu.VMEM((2, M_PER, HALF), jnp.bfloat16),
            pltpu.VMEM((N_DEV, 8, 128), jnp.float32),
            pltpu.SemaphoreType.DMA((2, 4)),
            pltpu.SemaphoreType.DMA((6,)),
            pltpu.SemaphoreType.DMA((3, 2)),
            pltpu.SemaphoreType.DMA((3,)),
            pltpu.SemaphoreType.DMA((3,)),
            pltpu.SemaphoreType.DMA((2,)),
            pltpu.SemaphoreType.DMA((2, 4)),
            pltpu.SemaphoreType.DMA((2, 4)),
        ],
        compiler_params=pltpu.CompilerParams(
            collective_id=0,
            vmem_limit_bytes=63 * 1024 * 1024,
        ),
    )(x, w_mat)
    return out


# device time: 175968 ns/iter; 1.0953x vs baseline; 1.0372x over previous
import jax
import jax.numpy as jnp
from jax import lax
from jax.experimental import pallas as pl
from jax.experimental.pallas import tpu as pltpu

_DIAG_NO_DOT = False

N_DEV = 4
M_PER = 1024
K = 4096
N_PER = 2048
HALF = 1024
N_STEPS = 8


def kernel(x, w_mat):
    def body(x_ref, w_hbm, out_hbm, stage_hbm, w_bufs, comm_bufs,
             amax_buf, w_sems, send_sems, recv_sems, ax_send_sems,
             ax_recv_sems, own_sems, epi_sems, epo_sems):
        my = lax.axis_index("i")

        barrier = pltpu.get_barrier_semaphore()
        for d in range(1, N_DEV):
            pl.semaphore_signal(
                barrier, inc=1,
                device_id=((my + d) % N_DEV,),
                device_id_type=pl.DeviceIdType.MESH,
            )
        pl.semaphore_wait(barrier, N_DEV - 1)

        def step_cols(t):
            dst = (my + 1 + t // 2) % N_DEV
            h = t % 2
            return dst, h

        def w_copy(t):
            dst, h = step_cols(t)
            col = dst * N_PER + h * HALF
            return [
                pltpu.make_async_copy(
                    w_hbm.at[pl.ds(j * (K // 4), K // 4), pl.ds(col, HALF)],
                    w_bufs.at[t % 2, pl.ds(j * (K // 4), K // 4)],
                    w_sems.at[t % 2, j],
                )
                for j in range(4)
            ]

        for c in w_copy(0):
            c.start()

        sends = {}
        own = []
        amax = jnp.float32(0.0)
        for t in range(N_STEPS):
            if t + 1 < N_STEPS:
                for c in w_copy(t + 1):
                    c.start()
            for c in w_copy(t):
                c.wait()
            if _DIAG_NO_DOT:
                blk = w_bufs[t % 2, pl.ds(0, M_PER)] * 2.0
            else:
                blk = jnp.dot(x_ref[...], w_bufs[t % 2],
                              preferred_element_type=jnp.float32)
            amax = jnp.maximum(amax, jnp.max(jnp.abs(blk)))
            dst, h = step_cols(t)
            if t - 4 in sends:
                sends[t - 4].wait_send()
            comm_bufs[t % 4] = blk.astype(jnp.bfloat16)
            if t < 6:
                rdma = pltpu.make_async_remote_copy(
                    src_ref=comm_bufs.at[t % 4],
                    dst_ref=stage_hbm.at[pl.ds(my * M_PER, M_PER),
                                         pl.ds(h * HALF, HALF)],
                    send_sem=send_sems.at[t],
                    recv_sem=recv_sems.at[t // 2, h],
                    device_id=(dst,),
                    device_id_type=pl.DeviceIdType.MESH,
                )
                rdma.start()
                sends[t] = rdma
            else:
                pass

        amax_buf[pl.ds(my, 1)] = jnp.full((1, 8, 128), amax, jnp.float32)
        ax_sends = []
        for d in range(1, N_DEV):
            r = pltpu.make_async_remote_copy(
                src_ref=amax_buf.at[pl.ds(my, 1)],
                dst_ref=amax_buf.at[pl.ds(my, 1)],
                send_sem=ax_send_sems.at[d - 1],
                recv_sem=ax_recv_sems.at[d - 1],
                device_id=((my + d) % N_DEV,),
                device_id_type=pl.DeviceIdType.MESH,
            )
            r.start()
            ax_sends.append(r)

        for cp in own:
            cp.wait()

        for d in range(1, N_DEV):
            src = (my - d) % N_DEV
            pltpu.make_async_remote_copy(
                src_ref=amax_buf.at[pl.ds(0, 1)],
                dst_ref=amax_buf.at[pl.ds(src, 1)],
                send_sem=ax_send_sems.at[d - 1],
                recv_sem=ax_recv_sems.at[d - 1],
                device_id=(0,),
                device_id_type=pl.DeviceIdType.MESH,
            ).wait_recv()
        for r in ax_sends:
            r.wait_send()

        g = jnp.max(amax_buf[...])
        scale = g / 448.0
        inv = 448.0 / g

        def chunk_rows(k):
            return ((my - k // 2) % N_DEV) * M_PER

        def recv_wait(k):
            b, h = k // 2, k % 2
            if b == 0:
                return
            pltpu.make_async_remote_copy(
                src_ref=comm_bufs.at[0],
                dst_ref=stage_hbm.at[pl.ds(chunk_rows(k), M_PER),
                                     pl.ds(h * HALF, HALF)],
                send_sem=send_sems.at[0],
                recv_sem=recv_sems.at[b - 1, h],
                device_id=(0,),
                device_id_type=pl.DeviceIdType.MESH,
            ).wait_recv()

        def epi_in(k):
            return [
                pltpu.make_async_copy(
                    stage_hbm.at[pl.ds(chunk_rows(k) + j * 256, 256),
                                 pl.ds((k % 2) * HALF, HALF)],
                    comm_bufs.at[2 + k % 2, pl.ds(j * 256, 256)],
                    epi_sems.at[k % 2, j],
                )
                for j in range(4)
            ]

        def epi_out(k):
            return [
                pltpu.make_async_copy(
                    w_bufs.at[k % 2, pl.ds(j * 256, 256)],
                    out_hbm.at[pl.ds(chunk_rows(k) + j * 256, 256),
                               pl.ds((k % 2) * HALF, HALF)],
                    epo_sems.at[k % 2, j],
                )
                for j in range(4)
            ]

        outs = {}
        for k in range(8):
            if 2 <= k + 1 < 8:
                recv_wait(k + 1)
                for c in epi_in(k + 1):
                    c.start()
            if k >= 2:
                for c in epi_in(k):
                    c.wait()
            if k - 2 in outs:
                for c in outs[k - 2]:
                    c.wait()
            y = comm_bufs[2 + k % 2].astype(jnp.float32)
            q = jnp.clip(y * inv, -448.0, 448.0)
            q = q.astype(jnp.float8_e4m3fn).astype(jnp.float32)
            w_bufs[k % 2, pl.ds(0, M_PER)] = q * scale
            cps = epi_out(k)
            for c in cps:
                c.start()
            outs[k] = cps
        for k in (6, 7):
            for c in outs[k]:
                c.wait()
        sends[4].wait_send()
        sends[5].wait_send()

    out, _ = pl.pallas_call(
        body,
        out_shape=(
            jax.ShapeDtypeStruct((N_DEV * M_PER, N_PER), jnp.float32),
            jax.ShapeDtypeStruct((N_DEV * M_PER, N_PER), jnp.bfloat16),
        ),
        in_specs=[
            pl.BlockSpec(memory_space=pltpu.VMEM),
            pl.BlockSpec(memory_space=pl.ANY),
        ],
        out_specs=(
            pl.BlockSpec(memory_space=pl.ANY),
            pl.BlockSpec(memory_space=pl.ANY),
        ),
        scratch_shapes=[
            pltpu.VMEM((2, K, HALF), jnp.float32),
            pltpu.VMEM((4, M_PER, HALF), jnp.bfloat16),
            pltpu.VMEM((N_DEV, 8, 128), jnp.float32),
            pltpu.SemaphoreType.DMA((2, 4)),
            pltpu.SemaphoreType.DMA((6,)),
            pltpu.SemaphoreType.DMA((3, 2)),
            pltpu.SemaphoreType.DMA((3,)),
            pltpu.SemaphoreType.DMA((3,)),
            pltpu.SemaphoreType.DMA((2,)),
            pltpu.SemaphoreType.DMA((2, 4)),
            pltpu.SemaphoreType.DMA((2, 4)),
        ],
        compiler_params=pltpu.CompilerParams(
            collective_id=0,
            vmem_limit_bytes=63 * 1024 * 1024,
        ),
    )(x, w_mat)
    return out
